# Initial kernel scaffold; baseline (speedup 1.0000x reference)
import jax
import jax.numpy as jnp
from jax.experimental import pallas as pl


def kernel(features, edge_index, W1, b1, W2, b2):
    raise NotImplementedError("write your pallas kernel here")


# speedup vs baseline: 4.8114x; 4.8114x over previous
import functools

import jax
import jax.numpy as jnp
from jax import lax
from jax.experimental import pallas as pl
from jax.experimental.pallas import tpu as pltpu
from jax.experimental.pallas import tpu_sc as plsc

N = 10000
E = 320000
D_IN = 128
D_H = 128
D_OUT = 64

NC = 2
NS = 16
NW = NC * NS

CHUNK = 128
N_PAD = 10240
ROWS_PER_TILE = N_PAD // NS
CHUNKS_PER_TILE = 79
EDGES_PER_TILE = CHUNKS_PER_TILE * CHUNK
E_PAD = NW * EDGES_PER_TILE

_MESH = plsc.VectorSubcoreMesh(
    core_axis_name="c", subcore_axis_name="s", num_cores=NC, num_subcores=NS)


def _deg_body(src_ref, dst_ref, ones_ref, zeros_ref, do_out, di_out,
              idx_v, ones_v, do_acc, di_acc):
    c = lax.axis_index("c")
    s = lax.axis_index("s")
    wid = c * NS + s
    row0 = s * ROWS_PER_TILE
    pltpu.sync_copy(zeros_ref.at[pl.ds(row0, ROWS_PER_TILE)],
                    do_acc.at[pl.ds(row0, ROWS_PER_TILE)])
    pltpu.sync_copy(zeros_ref.at[pl.ds(row0, ROWS_PER_TILE)],
                    di_acc.at[pl.ds(row0, ROWS_PER_TILE)])
    pltpu.sync_copy(ones_ref, ones_v)
    plsc.subcore_barrier()

    def step(i, carry):
        base = wid * EDGES_PER_TILE + i * CHUNK
        pltpu.sync_copy(src_ref.at[pl.ds(base, CHUNK)], idx_v)
        pltpu.sync_copy(ones_v, do_acc.at[idx_v], add=True)
        pltpu.sync_copy(dst_ref.at[pl.ds(base, CHUNK)], idx_v)
        pltpu.sync_copy(ones_v, di_acc.at[idx_v], add=True)
        return carry

    lax.fori_loop(0, CHUNKS_PER_TILE, step, 0)
    plsc.subcore_barrier()
    pltpu.sync_copy(do_acc.at[pl.ds(row0, ROWS_PER_TILE)],
                    do_out.at[c, pl.ds(row0, ROWS_PER_TILE)])
    pltpu.sync_copy(di_acc.at[pl.ds(row0, ROWS_PER_TILE)],
                    di_out.at[c, pl.ds(row0, ROWS_PER_TILE)])


def _make_deg_kernel():
    return pl.kernel(
        _deg_body,
        out_type=(jax.ShapeDtypeStruct((NC, N_PAD, 16), jnp.float32),
                  jax.ShapeDtypeStruct((NC, N_PAD, 16), jnp.float32)),
        mesh=_MESH,
        compiler_params=pltpu.CompilerParams(use_tc_tiling_on_sc=False),
        scratch_types=[
            pltpu.VMEM((CHUNK,), jnp.int32),
            pltpu.VMEM((CHUNK, 16), jnp.float32),
            pltpu.VMEM_SHARED((N_PAD, 16), jnp.float32),
            pltpu.VMEM_SHARED((N_PAD, 16), jnp.float32),
        ],
    )


def _prop_body(table_ref, src_ref, dst_ref, zeros_ref, out_ref,
               src_v, dst_v, rows_v, sem, acc):
    c = lax.axis_index("c")
    s = lax.axis_index("s")
    wid = c * NS + s
    row0 = s * ROWS_PER_TILE
    pltpu.sync_copy(zeros_ref.at[pl.ds(row0, ROWS_PER_TILE)],
                    acc.at[pl.ds(row0, ROWS_PER_TILE)])
    plsc.subcore_barrier()

    def step(i, carry):
        base = wid * EDGES_PER_TILE + i * CHUNK
        pltpu.sync_copy(src_ref.at[pl.ds(base, CHUNK)], src_v)
        pltpu.sync_copy(dst_ref.at[pl.ds(base, CHUNK)], dst_v)
        pltpu.async_copy(table_ref.at[src_v], rows_v, sem).wait()
        pltpu.sync_copy(rows_v, acc.at[dst_v], add=True)
        return carry

    lax.fori_loop(0, CHUNKS_PER_TILE, step, 0)
    plsc.subcore_barrier()
    pltpu.sync_copy(acc.at[pl.ds(row0, ROWS_PER_TILE)],
                    out_ref.at[c, pl.ds(row0, ROWS_PER_TILE)])


def _make_prop_kernel(d):
    return pl.kernel(
        _prop_body,
        out_type=jax.ShapeDtypeStruct((NC, N_PAD, d), jnp.float32),
        mesh=_MESH,
        compiler_params=pltpu.CompilerParams(use_tc_tiling_on_sc=False),
        scratch_types=[
            pltpu.VMEM((CHUNK,), jnp.int32),
            pltpu.VMEM((CHUNK,), jnp.int32),
            pltpu.VMEM((CHUNK, d), jnp.float32),
            pltpu.SemaphoreType.DMA,
            pltpu.VMEM_SHARED((N_PAD, d), jnp.float32),
        ],
    )


def _norm_from(parts_ref):
    deg = parts_ref[0, :, 0:1] + parts_ref[1, :, 0:1]
    return jnp.where(deg > 0.0, lax.rsqrt(jnp.maximum(deg, 1.0)), 0.0)


def _tc1_body(do_ref, feat_ref, w1_ref, y1_ref):
    nsrc = _norm_from(do_ref)
    y1_ref[...] = jnp.dot(feat_ref[...] * nsrc, w1_ref[...],
                          preferred_element_type=jnp.float32)


def _tc2_body(p_ref, do_ref, di_ref, b1_ref, w2_ref, y2_ref):
    nsrc = _norm_from(do_ref)
    ndst = _norm_from(di_ref)
    agg = (p_ref[0] + p_ref[1]) * ndst + b1_ref[...]
    h = jnp.maximum(agg, 0.0)
    y2_ref[...] = jnp.dot(h * nsrc, w2_ref[...],
                          preferred_element_type=jnp.float32)


def _tc3_body(p_ref, di_ref, b2_ref, o_ref):
    ndst = _norm_from(di_ref)
    o_ref[...] = (p_ref[0] + p_ref[1]) * ndst + b2_ref[...]


_R = 1024
_GRID = N_PAD // _R


def _deg_spec():
    return pl.BlockSpec((NC, _R, 16), lambda i: (0, i, 0))


def kernel(features, edge_index, W1, b1, W2, b2):
    src = edge_index[0]
    dst = edge_index[1]
    pad_idx = jnp.full((E_PAD - E,), N, dtype=jnp.int32)
    src_p = jnp.concatenate([src, pad_idx])
    dst_p = jnp.concatenate([dst, pad_idx])
    feat_p = jnp.zeros((N_PAD, D_IN), jnp.float32).at[:N].set(features)
    ones16 = jnp.ones((CHUNK, 16), jnp.float32)
    zeros16 = jnp.zeros((N_PAD, 16), jnp.float32)
    zeros_h = jnp.zeros((N_PAD, D_H), jnp.float32)
    zeros_o = jnp.zeros((N_PAD, D_OUT), jnp.float32)

    do_part, di_part = _make_deg_kernel()(src_p, dst_p, ones16, zeros16)

    y1 = pl.pallas_call(
        _tc1_body,
        grid=(_GRID,),
        in_specs=[
            _deg_spec(),
            pl.BlockSpec((_R, D_IN), lambda i: (i, 0)),
            pl.BlockSpec((D_IN, D_H), lambda i: (0, 0)),
        ],
        out_specs=pl.BlockSpec((_R, D_H), lambda i: (i, 0)),
        out_shape=jax.ShapeDtypeStruct((N_PAD, D_H), jnp.float32),
    )(do_part, feat_p, W1)

    p1 = _make_prop_kernel(D_H)(y1, src_p, dst_p, zeros_h)

    y2 = pl.pallas_call(
        _tc2_body,
        grid=(_GRID,),
        in_specs=[
            pl.BlockSpec((NC, _R, D_H), lambda i: (0, i, 0)),
            _deg_spec(),
            _deg_spec(),
            pl.BlockSpec((1, D_H), lambda i: (0, 0)),
            pl.BlockSpec((D_H, D_OUT), lambda i: (0, 0)),
        ],
        out_specs=pl.BlockSpec((_R, D_OUT), lambda i: (i, 0)),
        out_shape=jax.ShapeDtypeStruct((N_PAD, D_OUT), jnp.float32),
    )(p1, do_part, di_part, b1.reshape(1, D_H), W2)

    p2 = _make_prop_kernel(D_OUT)(y2, src_p, dst_p, zeros_o)

    out = pl.pallas_call(
        _tc3_body,
        grid=(_GRID,),
        in_specs=[
            pl.BlockSpec((NC, _R, D_OUT), lambda i: (0, i, 0)),
            _deg_spec(),
            pl.BlockSpec((1, D_OUT), lambda i: (0, 0)),
        ],
        out_specs=pl.BlockSpec((_R, D_OUT), lambda i: (i, 0)),
        out_shape=jax.ShapeDtypeStruct((N_PAD, D_OUT), jnp.float32),
    )(p2, di_part, b2.reshape(1, D_OUT))

    return out[:N]
